# baseline (speedup 1.0000x reference)
import functools

import jax
import jax.numpy as jnp
from jax import lax
from jax.experimental import pallas as pl
from jax.experimental.pallas import tpu as pltpu
from jax.experimental.pallas import tpu_sc as plsc

_B = 16384
_EMB = 16
_NC = 2
_NS = 16
_NW = _NC * _NS
_BPW = _B // _NW
_CH = 128
_NCH = _BPW // _CH

_mesh = plsc.VectorSubcoreMesh(core_axis_name="c", subcore_axis_name="s")


@functools.partial(
    pl.kernel,
    mesh=_mesh,
    out_type=[
        jax.ShapeDtypeStruct((_B, _EMB), jnp.float32),
        jax.ShapeDtypeStruct((_B, _EMB), jnp.float32),
        jax.ShapeDtypeStruct((_B, _EMB), jnp.float32),
    ],
    scratch_types=[
        pltpu.VMEM((_NCH, _CH), jnp.int32),
        pltpu.VMEM((_NCH, _CH), jnp.int32),
        pltpu.VMEM((_BPW, _EMB), jnp.float32),
        pltpu.VMEM((_BPW, _EMB), jnp.float32),
        pltpu.VMEM((_BPW, _EMB), jnp.float32),
        pltpu.VMEM((_BPW, _EMB), jnp.float32),
        pltpu.SemaphoreType.DMA,
    ],
    compiler_params=pltpu.CompilerParams(use_tc_tiling_on_sc=False),
)
def _sc_gather(uidx_hbm, iidx_hbm, gu_hbm, gi_hbm, mu_hbm, mi_hbm,
               xgmf_hbm, xum_hbm, xim_hbm,
               uidx_v, iidx_v, gu_v, gi_v, mu_v, mi_v, sem):
    wid = lax.axis_index("s") * _NC + lax.axis_index("c")
    base = wid * _BPW

    pltpu.sync_copy(uidx_hbm.at[wid], uidx_v)
    pltpu.sync_copy(iidx_hbm.at[wid], iidx_v)

    copies = []
    for table, idx_v, dst in (
        (gu_hbm, uidx_v, gu_v),
        (gi_hbm, iidx_v, gi_v),
        (mu_hbm, uidx_v, mu_v),
        (mi_hbm, iidx_v, mi_v),
    ):
        for j in range(_NCH):
            copies.append(
                pltpu.async_copy(
                    table.at[idx_v.at[j]], dst.at[pl.ds(j * _CH, _CH)], sem))
    for c in copies:
        c.wait()

    def body(i, carry):
        gu_v[i, :] = gu_v[i, :] * gi_v[i, :]
        return carry

    lax.fori_loop(0, _BPW, body, 0)

    pltpu.sync_copy(gu_v, xgmf_hbm.at[pl.ds(base, _BPW)])
    pltpu.sync_copy(mu_v, xum_hbm.at[pl.ds(base, _BPW)])
    pltpu.sync_copy(mi_v, xim_hbm.at[pl.ds(base, _BPW)])


def _dense_body(xgmf, xum, xim, gen, gW, gb, W1, b1, W2, b2, Wf, bf, out):
    xg = jnp.dot(gen[...], gW[...], preferred_element_type=jnp.float32) + gb[...]
    h = jnp.concatenate([xum[...], xim[...], xg], axis=1)
    h = jnp.maximum(
        jnp.dot(h, W1[...], preferred_element_type=jnp.float32) + b1[...], 0.0)
    h = jnp.maximum(
        jnp.dot(h, W2[...], preferred_element_type=jnp.float32) + b2[...], 0.0)
    wf = Wf[...]
    acc = jnp.dot(xgmf[...], wf[0:_EMB, :], preferred_element_type=jnp.float32)
    acc = acc + jnp.dot(h, wf[_EMB:, :], preferred_element_type=jnp.float32)
    out[...] = acc + bf[...]


_BT = 2048


def _dense(xgmf, xum, xim, gen, gW, gb, W1, b1, W2, b2, Wf, bf):
    grid = (_B // _BT,)
    row = lambda i: (i, 0)
    full = lambda i: (0, 0)
    return pl.pallas_call(
        _dense_body,
        grid=grid,
        in_specs=[
            pl.BlockSpec((_BT, _EMB), row),
            pl.BlockSpec((_BT, _EMB), row),
            pl.BlockSpec((_BT, _EMB), row),
            pl.BlockSpec((_BT, 18), row),
            pl.BlockSpec((18, 16), full),
            pl.BlockSpec((1, 16), full),
            pl.BlockSpec((48, 128), full),
            pl.BlockSpec((1, 128), full),
            pl.BlockSpec((128, 64), full),
            pl.BlockSpec((1, 64), full),
            pl.BlockSpec((80, 1), full),
            pl.BlockSpec((1, 1), full),
        ],
        out_specs=pl.BlockSpec((_BT, 1), row),
        out_shape=jax.ShapeDtypeStruct((_B, 1), jnp.float32),
        compiler_params=pltpu.CompilerParams(
            dimension_semantics=("parallel",)),
    )(xgmf, xum, xim, gen, gW, gb, W1, b1, W2, b2, Wf, bf)


def kernel(user_indices, item_indices, genres_vec, gmf_user_emb, gmf_item_emb,
           mlp_user_emb, mlp_item_emb, genres_W, genres_b, W1, b1, W2, b2,
           Wf, bf):
    u3 = user_indices.astype(jnp.int32).reshape(_NW, _NCH, _CH)
    i3 = item_indices.astype(jnp.int32).reshape(_NW, _NCH, _CH)
    x_gmf, xu_mlp, xi_mlp = _sc_gather(
        u3, i3, gmf_user_emb, gmf_item_emb, mlp_user_emb, mlp_item_emb)
    out = _dense(
        x_gmf, xu_mlp, xi_mlp, genres_vec, genres_W,
        genres_b.reshape(1, -1), W1, b1.reshape(1, -1), W2,
        b2.reshape(1, -1), Wf, bf.reshape(1, -1))
    return out[:, 0]
